# baseline (speedup 1.0000x reference)
import jax
import jax.numpy as jnp
from jax import lax
from jax.experimental import pallas as pl
from jax.experimental.pallas import tpu as pltpu
from jax.experimental.pallas import tpu_sc as plsc

N_USERS = 50000
N_ITEMS = 50000
N_NODES = N_USERS + N_ITEMS
N_EDGES = 1600000
D = 32
BATCH = 16384

NC = 2
NS = 16
SUB = 128
NSUB_MAX = 4
CHUNK = SUB * NSUB_MAX
PAIRS = 100
CHUNKS_PER_TILE = 2 * PAIRS
E_PAD = NS * CHUNKS_PER_TILE * CHUNK
EDGES_PER_TILE = CHUNKS_PER_TILE * CHUNK
CBUF = CHUNK + SUB

HALF = N_NODES // NC
DUMMY = HALF
ACC_ROWS = HALF + 8
STRIPE = 3128
LAST_STRIPE = HALF - (NS - 1) * STRIPE

_mesh = plsc.VectorSubcoreMesh(
    core_axis_name="c", subcore_axis_name="s", num_cores=NC, num_subcores=NS
)


def _layer_body(emb_in, src_hbm, dst_hbm, val_hbm, zeros, emb_out,
                src_raw, dst_raw, val_raw, csrc, cdstf, cval, cdst2,
                rows_v, acc_sh, idx_sem, gat_sem, sct_sem):
    c = lax.axis_index("c")
    s = lax.axis_index("s")
    c0 = c * HALF
    ebase = s * EDGES_PER_TILE

    @pl.when(s < NS - 1)
    def _zero_full():
        pltpu.sync_copy(zeros, acc_sh.at[pl.ds(s * STRIPE, STRIPE)])

    @pl.when(s == NS - 1)
    def _zero_last():
        pltpu.sync_copy(zeros.at[pl.ds(0, LAST_STRIPE)],
                        acc_sh.at[pl.ds(s * STRIPE, LAST_STRIPE)])

    plsc.subcore_barrier()

    def fire_idx(i, b):
        off = ebase + i * CHUNK
        pltpu.async_copy(src_hbm.at[pl.ds(off, CHUNK)], src_raw.at[b], idx_sem)
        pltpu.async_copy(dst_hbm.at[pl.ds(off, CHUNK)], dst_raw.at[b], idx_sem)
        pltpu.async_copy(val_hbm.at[pl.ds(off, CHUNK)], val_raw.at[b], idx_sem)

    def wait_idx(b):
        pltpu.make_async_copy(src_hbm.at[pl.ds(0, CHUNK)], src_raw.at[b],
                              idx_sem).wait()
        pltpu.make_async_copy(dst_hbm.at[pl.ds(0, CHUNK)], dst_raw.at[b],
                              idx_sem).wait()
        pltpu.make_async_copy(val_hbm.at[pl.ds(0, CHUNK)], val_raw.at[b],
                              idx_sem).wait()

    fire_idx(0, 0)

    def one_chunk(i, b, prev_nsub):
        wait_idx(b)

        def grp(g, cnt):
            d = dst_raw[b, pl.ds(g * 16, 16)]
            sv = src_raw[b, pl.ds(g * 16, 16)]
            vv = val_raw[b, pl.ds(g * 16, 16)]
            loc = d - c0
            ok = (d >= c0) & (loc < HALF)
            plsc.store_compressed(csrc.at[pl.ds(cnt, 16)], sv, mask=ok)
            plsc.store_compressed(cdstf.at[pl.ds(cnt, 16)], loc, mask=ok)
            plsc.store_compressed(cval.at[pl.ds(cnt, 16)], vv, mask=ok)
            return cnt + plsc.all_reduce_population_count(ok)[0]

        cnt = lax.fori_loop(0, CHUNK // 16, grp, jnp.int32(0))

        @pl.when(i + 1 < CHUNKS_PER_TILE)
        def _prefetch():
            fire_idx(i + 1, 1 - b)

        zi = jnp.zeros((16,), jnp.int32)
        zdum = jnp.full((16,), DUMMY, jnp.int32)
        zf = jnp.zeros((16,), jnp.float32)
        for t in range(SUB // 16):
            csrc[pl.ds(cnt + t * 16, 16)] = zi
            cdstf[pl.ds(cnt + t * 16, 16)] = zdum
            cval[pl.ds(cnt + t * 16, 16)] = zf
        nsub = (cnt + SUB - 1) // SUB

        def drain_sct(k, _):
            pltpu.make_async_copy(rows_v.at[k], acc_sh.at[cdst2.at[k]],
                                  sct_sem).wait()
            return 0

        lax.fori_loop(0, prev_nsub, drain_sct, 0)

        for j in range(NSUB_MAX):
            for k in range(SUB // 16):
                cdst2[j, pl.ds(k * 16, 16)] = cdstf[pl.ds((j * 8 + k) * 16, 16)]

        def fire_gat(j, _):
            pltpu.async_copy(emb_in.at[csrc.at[pl.ds(j * SUB, SUB)]],
                             rows_v.at[j], gat_sem)
            return 0

        lax.fori_loop(0, nsub, fire_gat, 0)

        def drain_gat(j, _):
            pltpu.make_async_copy(emb_in.at[csrc.at[pl.ds(j * SUB, SUB)]],
                                  rows_v.at[j], gat_sem).wait()
            return 0

        lax.fori_loop(0, nsub, drain_gat, 0)

        def scale_sub(j, _):
            def scale_body(g, _2):
                vv = cval[pl.ds(j * SUB + g * 16, 16)]
                for i2 in range(16):
                    e = g * 16 + i2
                    vi = vv[i2]
                    rows_v[j, e, pl.ds(0, 16)] = rows_v[j, e, pl.ds(0, 16)] * vi
                    rows_v[j, e, pl.ds(16, 16)] = (
                        rows_v[j, e, pl.ds(16, 16)] * vi)
                return 0

            lax.fori_loop(0, SUB // 16, scale_body, 0)
            pltpu.async_copy(rows_v.at[j], acc_sh.at[cdst2.at[j]],
                             sct_sem, add=True)
            return 0

        lax.fori_loop(0, nsub, scale_sub, 0)
        return nsub

    def pair_body(p, prev_nsub):
        prev_nsub = one_chunk(2 * p, 0, prev_nsub)
        prev_nsub = one_chunk(2 * p + 1, 1, prev_nsub)
        return prev_nsub

    last_nsub = lax.fori_loop(0, PAIRS, pair_body, jnp.int32(0))

    def drain_last(k, _):
        pltpu.make_async_copy(rows_v.at[k], acc_sh.at[cdst2.at[k]],
                              sct_sem).wait()
        return 0

    lax.fori_loop(0, last_nsub, drain_last, 0)

    plsc.subcore_barrier()

    @pl.when(s < NS - 1)
    def _wb_full():
        pltpu.sync_copy(acc_sh.at[pl.ds(s * STRIPE, STRIPE)],
                        emb_out.at[pl.ds(c0 + s * STRIPE, STRIPE)])

    @pl.when(s == NS - 1)
    def _wb_last():
        pltpu.sync_copy(acc_sh.at[pl.ds(s * STRIPE, LAST_STRIPE)],
                        emb_out.at[pl.ds(c0 + s * STRIPE, LAST_STRIPE)])


_params = pltpu.CompilerParams(use_tc_tiling_on_sc=False,
                               needs_layout_passes=False)

_layer = pl.kernel(
    _layer_body,
    out_type=jax.ShapeDtypeStruct((N_NODES, D), jnp.float32),
    mesh=_mesh,
    compiler_params=_params,
    scratch_types=[
        pltpu.VMEM((2, CHUNK), jnp.int32),
        pltpu.VMEM((2, CHUNK), jnp.int32),
        pltpu.VMEM((2, CHUNK), jnp.float32),
        pltpu.VMEM((CBUF,), jnp.int32),
        pltpu.VMEM((CBUF,), jnp.int32),
        pltpu.VMEM((CBUF,), jnp.float32),
        pltpu.VMEM((NSUB_MAX, SUB), jnp.int32),
        pltpu.VMEM((NSUB_MAX, SUB, D), jnp.float32),
        pltpu.VMEM_SHARED((ACC_ROWS, D), jnp.float32),
        pltpu.SemaphoreType.DMA,
        pltpu.SemaphoreType.DMA,
        pltpu.SemaphoreType.DMA,
    ],
)

B_PER_W = BATCH // (NC * NS)
BROWS_PER_W = B_PER_W // SUB


def _final_body(emb0, emb1, emb2, emb3, users2d, pos2d,
                ue, pe, uf, pf, idx_v, acc_v, sem):
    c = lax.axis_index("c")
    s = lax.axis_index("s")
    wid = s * NC + c
    row0 = wid * BROWS_PER_W
    base = wid * B_PER_W

    def lookup(idx2d, offset, out_raw, out_final):
        pltpu.async_copy(idx2d.at[pl.ds(row0, BROWS_PER_W)], idx_v, sem).wait()
        if offset:
            for j in range(BROWS_PER_W):
                for k in range(SUB // 16):
                    idx_v[j, pl.ds(k * 16, 16)] = (
                        idx_v[j, pl.ds(k * 16, 16)] + offset)
        for j in range(BROWS_PER_W):
            pltpu.async_copy(emb0.at[idx_v.at[j]],
                             acc_v.at[pl.ds(j * SUB, SUB)], sem).wait()
        pltpu.sync_copy(acc_v, out_raw.at[pl.ds(base, B_PER_W)])
        for emb in (emb1, emb2, emb3):
            for j in range(BROWS_PER_W):
                pltpu.async_copy(emb.at[idx_v.at[j]],
                                 acc_v.at[pl.ds(j * SUB, SUB)], sem,
                                 add=True).wait()

        def scale_body(i, _):
            acc_v[i, pl.ds(0, 16)] = acc_v[i, pl.ds(0, 16)] * 0.25
            acc_v[i, pl.ds(16, 16)] = acc_v[i, pl.ds(16, 16)] * 0.25
            return 0

        lax.fori_loop(0, B_PER_W, scale_body, 0)
        pltpu.sync_copy(acc_v, out_final.at[pl.ds(base, B_PER_W)])

    lookup(users2d, 0, ue, uf)
    lookup(pos2d, N_USERS, pe, pf)


_final = pl.kernel(
    _final_body,
    out_type=(
        jax.ShapeDtypeStruct((BATCH, D), jnp.float32),
        jax.ShapeDtypeStruct((BATCH, D), jnp.float32),
        jax.ShapeDtypeStruct((BATCH, D), jnp.float32),
        jax.ShapeDtypeStruct((BATCH, D), jnp.float32),
    ),
    mesh=_mesh,
    compiler_params=_params,
    scratch_types=[
        pltpu.VMEM((BROWS_PER_W, SUB), jnp.int32),
        pltpu.VMEM((B_PER_W, D), jnp.float32),
        pltpu.SemaphoreType.DMA,
    ],
)


def kernel(user_table, item_table, edge_val, edge_src, edge_dst, users, pos_items):
    emb0 = jnp.concatenate([user_table, item_table], axis=0)
    pad = E_PAD - N_EDGES
    src_p = jnp.concatenate(
        [edge_src.astype(jnp.int32), jnp.zeros((pad,), jnp.int32)])
    dst_p = jnp.concatenate(
        [edge_dst.astype(jnp.int32), jnp.full((pad,), N_NODES, jnp.int32)])
    val_p = jnp.concatenate([edge_val, jnp.zeros((pad,), jnp.float32)])
    zeros = jnp.zeros((STRIPE, D), jnp.float32)

    e1 = _layer(emb0, src_p, dst_p, val_p, zeros)
    e2 = _layer(e1, src_p, dst_p, val_p, zeros)
    e3 = _layer(e2, src_p, dst_p, val_p, zeros)

    users2d = users.astype(jnp.int32).reshape(BATCH // SUB, SUB)
    pos2d = pos_items.astype(jnp.int32).reshape(BATCH // SUB, SUB)
    return _final(emb0, e1, e2, e3, users2d, pos2d)


# speedup vs baseline: 3.3202x; 3.3202x over previous
import jax
import jax.numpy as jnp
from jax import lax
from jax.experimental import pallas as pl
from jax.experimental.pallas import tpu as pltpu
from jax.experimental.pallas import tpu_sc as plsc

N_USERS = 50000
N_ITEMS = 50000
N_NODES = N_USERS + N_ITEMS
N_EDGES = 1600000
D = 32
BATCH = 16384

NC = 2
NS = 16
SUB = 128
CHUNK_ROWS = 16
CHUNK = SUB * CHUNK_ROWS
E_PAD = ((N_EDGES + NS * CHUNK - 1) // (NS * CHUNK)) * (NS * CHUNK)
ROWS = E_PAD // SUB
ROWS_PER_TILE = ROWS // NS
CHUNKS_PER_TILE = ROWS_PER_TILE // CHUNK_ROWS

HALF = N_NODES // NC
DUMMY = HALF
ACC_ROWS = HALF + 8
STRIPE = 3128
LAST_STRIPE = HALF - (NS - 1) * STRIPE

_mesh = plsc.VectorSubcoreMesh(
    core_axis_name="c", subcore_axis_name="s", num_cores=NC, num_subcores=NS
)


def _layer_body(emb_in, src2d, dst2d, val2d, zeros, emb_out,
                src_v, dst_v, val_v, rows_v, acc_sh, sem):
    c = lax.axis_index("c")
    s = lax.axis_index("s")
    c0 = c * HALF

    @pl.when(s < NS - 1)
    def _zero_full():
        pltpu.sync_copy(zeros, acc_sh.at[pl.ds(s * STRIPE, STRIPE)])

    @pl.when(s == NS - 1)
    def _zero_last():
        pltpu.sync_copy(zeros.at[pl.ds(0, LAST_STRIPE)],
                        acc_sh.at[pl.ds(s * STRIPE, LAST_STRIPE)])

    plsc.subcore_barrier()

    def chunk_body(i, carry):
        off = s * ROWS_PER_TILE + i * CHUNK_ROWS
        pltpu.async_copy(src2d.at[pl.ds(off, CHUNK_ROWS)], src_v, sem).wait()
        pltpu.async_copy(dst2d.at[pl.ds(off, CHUNK_ROWS)], dst_v, sem).wait()
        pltpu.async_copy(val2d.at[pl.ds(off, CHUNK_ROWS)], val_v, sem).wait()

        for j in range(CHUNK_ROWS):
            for k in range(SUB // 16):
                d = dst_v[j, pl.ds(k * 16, 16)]
                loc = d - c0
                ok = (d >= c0) & (loc < HALF)
                dst_v[j, pl.ds(k * 16, 16)] = jnp.where(ok, loc, DUMMY)

        for j in range(CHUNK_ROWS):
            pltpu.async_copy(emb_in.at[src_v.at[j]], rows_v, sem).wait()

            def scale_body(g, _):
                vv = val_v[j, pl.ds(g * 16, 16)]
                for i in range(16):
                    e = g * 16 + i
                    vi = vv[i]
                    rows_v[e, pl.ds(0, 16)] = rows_v[e, pl.ds(0, 16)] * vi
                    rows_v[e, pl.ds(16, 16)] = rows_v[e, pl.ds(16, 16)] * vi
                return 0

            lax.fori_loop(0, SUB // 16, scale_body, 0)
            pltpu.async_copy(rows_v, acc_sh.at[dst_v.at[j]], sem, add=True).wait()
        return carry

    lax.fori_loop(0, CHUNKS_PER_TILE, chunk_body, 0)

    plsc.subcore_barrier()

    @pl.when(s < NS - 1)
    def _wb_full():
        pltpu.sync_copy(acc_sh.at[pl.ds(s * STRIPE, STRIPE)],
                        emb_out.at[pl.ds(c0 + s * STRIPE, STRIPE)])

    @pl.when(s == NS - 1)
    def _wb_last():
        pltpu.sync_copy(acc_sh.at[pl.ds(s * STRIPE, LAST_STRIPE)],
                        emb_out.at[pl.ds(c0 + s * STRIPE, LAST_STRIPE)])


_params = pltpu.CompilerParams(use_tc_tiling_on_sc=False,
                               needs_layout_passes=False)

_layer = pl.kernel(
    _layer_body,
    out_type=jax.ShapeDtypeStruct((N_NODES, D), jnp.float32),
    mesh=_mesh,
    compiler_params=_params,
    scratch_types=[
        pltpu.VMEM((CHUNK_ROWS, SUB), jnp.int32),
        pltpu.VMEM((CHUNK_ROWS, SUB), jnp.int32),
        pltpu.VMEM((CHUNK_ROWS, SUB), jnp.float32),
        pltpu.VMEM((SUB, D), jnp.float32),
        pltpu.VMEM_SHARED((ACC_ROWS, D), jnp.float32),
        pltpu.SemaphoreType.DMA,
    ],
)

B_PER_W = BATCH // (NC * NS)
BROWS_PER_W = B_PER_W // SUB


def _final_body(emb0, emb1, emb2, emb3, users2d, pos2d,
                ue, pe, uf, pf, idx_v, acc_v, sem):
    c = lax.axis_index("c")
    s = lax.axis_index("s")
    wid = s * NC + c
    row0 = wid * BROWS_PER_W
    base = wid * B_PER_W

    def lookup(idx2d, offset, out_raw, out_final):
        pltpu.async_copy(idx2d.at[pl.ds(row0, BROWS_PER_W)], idx_v, sem).wait()
        if offset:
            for j in range(BROWS_PER_W):
                for k in range(SUB // 16):
                    idx_v[j, pl.ds(k * 16, 16)] = (
                        idx_v[j, pl.ds(k * 16, 16)] + offset)
        for j in range(BROWS_PER_W):
            pltpu.async_copy(emb0.at[idx_v.at[j]],
                             acc_v.at[pl.ds(j * SUB, SUB)], sem).wait()
        pltpu.sync_copy(acc_v, out_raw.at[pl.ds(base, B_PER_W)])
        for emb in (emb1, emb2, emb3):
            for j in range(BROWS_PER_W):
                pltpu.async_copy(emb.at[idx_v.at[j]],
                                 acc_v.at[pl.ds(j * SUB, SUB)], sem,
                                 add=True).wait()

        def scale_body(i, _):
            acc_v[i, pl.ds(0, 16)] = acc_v[i, pl.ds(0, 16)] * 0.25
            acc_v[i, pl.ds(16, 16)] = acc_v[i, pl.ds(16, 16)] * 0.25
            return 0

        lax.fori_loop(0, B_PER_W, scale_body, 0)
        pltpu.sync_copy(acc_v, out_final.at[pl.ds(base, B_PER_W)])

    lookup(users2d, 0, ue, uf)
    lookup(pos2d, N_USERS, pe, pf)


_final = pl.kernel(
    _final_body,
    out_type=(
        jax.ShapeDtypeStruct((BATCH, D), jnp.float32),
        jax.ShapeDtypeStruct((BATCH, D), jnp.float32),
        jax.ShapeDtypeStruct((BATCH, D), jnp.float32),
        jax.ShapeDtypeStruct((BATCH, D), jnp.float32),
    ),
    mesh=_mesh,
    compiler_params=_params,
    scratch_types=[
        pltpu.VMEM((BROWS_PER_W, SUB), jnp.int32),
        pltpu.VMEM((B_PER_W, D), jnp.float32),
        pltpu.SemaphoreType.DMA,
    ],
)


def kernel(user_table, item_table, edge_val, edge_src, edge_dst, users, pos_items):
    emb0 = jnp.concatenate([user_table, item_table], axis=0)
    pad = E_PAD - N_EDGES
    src2d = jnp.concatenate(
        [edge_src.astype(jnp.int32), jnp.zeros((pad,), jnp.int32)]
    ).reshape(ROWS, SUB)
    dst2d = jnp.concatenate(
        [edge_dst.astype(jnp.int32), jnp.zeros((pad,), jnp.int32)]
    ).reshape(ROWS, SUB)
    val2d = jnp.concatenate(
        [edge_val, jnp.zeros((pad,), jnp.float32)]
    ).reshape(ROWS, SUB)
    zeros = jnp.zeros((STRIPE, D), jnp.float32)

    e1 = _layer(emb0, src2d, dst2d, val2d, zeros)
    e2 = _layer(e1, src2d, dst2d, val2d, zeros)
    e3 = _layer(e2, src2d, dst2d, val2d, zeros)

    users2d = users.astype(jnp.int32).reshape(BATCH // SUB, SUB)
    pos2d = pos_items.astype(jnp.int32).reshape(BATCH // SUB, SUB)
    return _final(emb0, e1, e2, e3, users2d, pos2d)
